# baseline (speedup 1.0000x reference)
import functools

import jax
import jax.numpy as jnp
from jax import lax
from jax.experimental import pallas as pl
from jax.experimental.pallas import tpu as pltpu
from jax.experimental.pallas import tpu_sc as plsc

_HIDDEN = 1024
_NC = 2
_NS = 16
_NW = _NC * _NS
_CHUNK = 32


@functools.cache
def _sc_lookup(total: int):
    per_w = total // _NW
    nch = per_w // _CHUNK
    mesh = plsc.VectorSubcoreMesh(core_axis_name="c", subcore_axis_name="s")

    @functools.partial(
        pl.kernel,
        out_type=jax.ShapeDtypeStruct((total, _HIDDEN), jnp.float32),
        mesh=mesh,
        scratch_types=[
            pltpu.VMEM((nch, _CHUNK), jnp.int32),
            pltpu.VMEM((_CHUNK, _HIDDEN), jnp.float32),
            pltpu.VMEM((_CHUNK, _HIDDEN), jnp.float32),
            pltpu.SemaphoreType.DMA,
            pltpu.SemaphoreType.DMA,
        ],
    )
    def k(ids_hbm, table_hbm, out_hbm, idx_v, buf0, buf1, sem0, sem1):
        wid = lax.axis_index("s") * _NC + lax.axis_index("c")
        base = wid * per_w
        pltpu.sync_copy(ids_hbm.at[wid], idx_v)
        bufs = (buf0, buf1)
        sems = (sem0, sem1)
        copies = [None] * nch
        copies[0] = pltpu.async_copy(table_hbm.at[idx_v.at[0]], bufs[0], sems[0])
        for c in range(nch):
            if c + 1 < nch:
                copies[c + 1] = pltpu.async_copy(
                    table_hbm.at[idx_v.at[c + 1]], bufs[(c + 1) % 2], sems[(c + 1) % 2]
                )
            copies[c].wait()
            pltpu.sync_copy(bufs[c % 2], out_hbm.at[pl.ds(base + c * _CHUNK, _CHUNK)])

    return k


def kernel(ids, indicator_embs):
    b, s = ids.shape
    total = b * s
    ids_w = ids.astype(jnp.int32).reshape(_NW, total // _NW // _CHUNK, _CHUNK)
    out = _sc_lookup(total)(ids_w, indicator_embs)
    return out.reshape(b, s, _HIDDEN)


# speedup vs baseline: 1.0007x; 1.0007x over previous
import functools

import jax
import jax.numpy as jnp
from jax import lax
from jax.experimental import pallas as pl
from jax.experimental.pallas import tpu as pltpu
from jax.experimental.pallas import tpu_sc as plsc

_HIDDEN = 1024
_NC = 2
_NS = 16
_NW = _NC * _NS
_CHUNK = 16
_NBUF = 6
_LAG = 2


@functools.cache
def _sc_lookup(total: int):
    per_w = total // _NW
    nch = per_w // _CHUNK
    mesh = plsc.VectorSubcoreMesh(core_axis_name="c", subcore_axis_name="s")

    @functools.partial(
        pl.kernel,
        out_type=jax.ShapeDtypeStruct((total, _HIDDEN), jnp.float32),
        mesh=mesh,
        scratch_types=[
            pltpu.VMEM((nch, _CHUNK), jnp.int32),
            *[pltpu.VMEM((_CHUNK, _HIDDEN), jnp.float32) for _ in range(_NBUF)],
            *[pltpu.SemaphoreType.DMA for _ in range(2 * _NBUF)],
        ],
    )
    def k(ids_hbm, table_hbm, out_hbm, idx_v, *rest):
        bufs = rest[:_NBUF]
        gsems = rest[_NBUF:2 * _NBUF]
        ssems = rest[2 * _NBUF:]
        wid = lax.axis_index("s") * _NC + lax.axis_index("c")
        base = wid * per_w
        pltpu.sync_copy(ids_hbm.at[wid], idx_v)
        gcp = [None] * nch
        scp = [None] * nch
        for c in range(nch + _LAG):
            if c < nch:
                slot = c % _NBUF
                if c >= _NBUF:
                    scp[c - _NBUF].wait()
                gcp[c] = pltpu.async_copy(
                    table_hbm.at[idx_v.at[c]], bufs[slot], gsems[slot]
                )
            d = c - _LAG
            if d >= 0:
                gcp[d].wait()
                scp[d] = pltpu.async_copy(
                    bufs[d % _NBUF],
                    out_hbm.at[pl.ds(base + d * _CHUNK, _CHUNK)],
                    ssems[d % _NBUF],
                )
        for d in range(nch - _NBUF, nch):
            scp[d].wait()

    return k


def kernel(ids, indicator_embs):
    b, s = ids.shape
    total = b * s
    ids_w = ids.astype(jnp.int32).reshape(_NW, total // _NW // _CHUNK, _CHUNK)
    out = _sc_lookup(total)(ids_w, indicator_embs)
    return out.reshape(b, s, _HIDDEN)
